# initial kernel scaffold (unmeasured)
import jax
import jax.numpy as jnp
from jax import lax
from jax.experimental import pallas as pl
from jax.experimental.pallas import tpu as pltpu

N_DEV = 4
B_LOC = 2
SQ = 256
HQ = 16
DH = 64
D_MODEL = 512
D_QK = HQ * DH
GCOLS = D_QK // N_DEV
BLK = 64


def kernel(x, Wq, K_ext, V_ext, Wo):
    def body(x_ref, wq_ref, k_hbm, v_hbm, wo_ref, out_ref,
             wq_g, wo_g, k_vmem, v_vmem, x_bf, q_buf, ctx_buf,
             wq_send_sems, wo_send_sems, wq_recv_sems, wo_recv_sems,
             k_sem, v_sem):
        my = lax.axis_index("i")

        b0 = my * B_LOC
        k_copy = pltpu.make_async_copy(k_hbm.at[pl.ds(b0, B_LOC)], k_vmem, k_sem)
        v_copy = pltpu.make_async_copy(v_hbm.at[pl.ds(b0, B_LOC)], v_vmem, v_sem)
        k_copy.start()
        v_copy.start()

        barrier = pltpu.get_barrier_semaphore()
        for d in range(1, N_DEV):
            pl.semaphore_signal(
                barrier, inc=1,
                device_id=((my + d) % N_DEV,),
                device_id_type=pl.DeviceIdType.MESH,
            )
        pl.semaphore_wait(barrier, N_DEV - 1)

        for g in range(N_DEV):
            @pl.when(my == g)
            def _(g=g):
                wq_g[g] = wq_ref[...].astype(jnp.bfloat16)
                wo_g[g] = wo_ref[...].astype(jnp.bfloat16)
                for d in range(1, N_DEV):
                    peer = (g + d) % N_DEV
                    pltpu.make_async_remote_copy(
                        src_ref=wq_g.at[g], dst_ref=wq_g.at[g],
                        send_sem=wq_send_sems.at[d - 1],
                        recv_sem=wq_recv_sems.at[g],
                        device_id=(peer,),
                        device_id_type=pl.DeviceIdType.MESH,
                    ).start()
                    pltpu.make_async_remote_copy(
                        src_ref=wo_g.at[g], dst_ref=wo_g.at[g],
                        send_sem=wo_send_sems.at[d - 1],
                        recv_sem=wo_recv_sems.at[g],
                        device_id=(peer,),
                        device_id_type=pl.DeviceIdType.MESH,
                    ).start()

        x_bf[...] = x_ref[...].reshape(B_LOC * SQ, D_MODEL).astype(jnp.bfloat16)

        k_copy.wait()
        v_copy.wait()

        for g in range(N_DEV):
            @pl.when(my != g)
            def _(g=g):
                pltpu.make_async_remote_copy(
                    src_ref=wq_g.at[g], dst_ref=wq_g.at[g],
                    send_sem=wq_send_sems.at[0],
                    recv_sem=wq_recv_sems.at[g],
                    device_id=(g,), device_id_type=pl.DeviceIdType.MESH,
                ).wait_recv()
                pltpu.make_async_remote_copy(
                    src_ref=wo_g.at[g], dst_ref=wo_g.at[g],
                    send_sem=wo_send_sems.at[0],
                    recv_sem=wo_recv_sems.at[g],
                    device_id=(g,), device_id_type=pl.DeviceIdType.MESH,
                ).wait_recv()

        for g in range(N_DEV):
            q = jnp.dot(x_bf[...], wq_g[g], preferred_element_type=jnp.float32)
            q_buf[:, g * GCOLS:(g + 1) * GCOLS] = q.astype(jnp.bfloat16)

        rows = lax.broadcasted_iota(jnp.int32, (SQ, SQ), 0) // BLK
        cols = lax.broadcasted_iota(jnp.int32, (SQ, SQ), 1) // BLK
        mask = rows == cols
        for b in range(B_LOC):
            for h in range(HQ):
                qh = q_buf[b * SQ:(b + 1) * SQ, h * DH:(h + 1) * DH]
                kh = k_vmem[b, :, h, :].astype(jnp.bfloat16)
                s = lax.dot_general(
                    qh, kh, (((1,), (1,)), ((), ())),
                    preferred_element_type=jnp.float32,
                ) * 0.125
                s = jnp.where(mask, s, -1e9)
                m = jnp.max(s, axis=-1, keepdims=True)
                w = jnp.exp(s - m)
                w = w / jnp.sum(w, axis=-1, keepdims=True)
                vh = v_vmem[b, :, h, :].astype(jnp.bfloat16)
                ctx = jnp.dot(w.astype(jnp.bfloat16), vh,
                              preferred_element_type=jnp.float32)
                ctx_buf[b * SQ:(b + 1) * SQ, h * DH:(h + 1) * DH] = (
                    ctx.astype(jnp.bfloat16))

        acc = jnp.zeros((B_LOC * SQ, D_MODEL), jnp.float32)
        for g in range(N_DEV):
            acc += jnp.dot(ctx_buf[:, g * GCOLS:(g + 1) * GCOLS], wo_g[g],
                           preferred_element_type=jnp.float32)
        for b in range(B_LOC):
            out_ref[b, :, :] = acc[b * SQ:(b + 1) * SQ, :]

        for g in range(N_DEV):
            @pl.when(my == g)
            def _(g=g):
                for d in range(1, N_DEV):
                    peer = (g + d) % N_DEV
                    pltpu.make_async_remote_copy(
                        src_ref=wq_g.at[g], dst_ref=wq_g.at[g],
                        send_sem=wq_send_sems.at[d - 1],
                        recv_sem=wq_recv_sems.at[g],
                        device_id=(peer,),
                        device_id_type=pl.DeviceIdType.MESH,
                    ).wait_send()
                    pltpu.make_async_remote_copy(
                        src_ref=wo_g.at[g], dst_ref=wo_g.at[g],
                        send_sem=wo_send_sems.at[d - 1],
                        recv_sem=wo_recv_sems.at[g],
                        device_id=(peer,),
                        device_id_type=pl.DeviceIdType.MESH,
                    ).wait_send()

    any_space = getattr(pltpu, "ANY", None)
    if any_space is None:
        any_space = pltpu.MemorySpace.ANY

    return pl.pallas_call(
        body,
        out_shape=jax.ShapeDtypeStruct((B_LOC, SQ, D_MODEL), jnp.float32),
        in_specs=[
            pl.BlockSpec(memory_space=pltpu.VMEM),
            pl.BlockSpec(memory_space=pltpu.VMEM),
            pl.BlockSpec(memory_space=any_space),
            pl.BlockSpec(memory_space=any_space),
            pl.BlockSpec(memory_space=pltpu.VMEM),
        ],
        out_specs=pl.BlockSpec(memory_space=pltpu.VMEM),
        scratch_shapes=[
            pltpu.VMEM((N_DEV, D_MODEL, GCOLS), jnp.bfloat16),
            pltpu.VMEM((N_DEV, GCOLS, D_MODEL), jnp.bfloat16),
            pltpu.VMEM((B_LOC, SQ, HQ, DH), jnp.float32),
            pltpu.VMEM((B_LOC, SQ, HQ, DH), jnp.float32),
            pltpu.VMEM((B_LOC * SQ, D_MODEL), jnp.bfloat16),
            pltpu.VMEM((B_LOC * SQ, D_QK), jnp.bfloat16),
            pltpu.VMEM((B_LOC * SQ, D_QK), jnp.bfloat16),
            pltpu.SemaphoreType.DMA((N_DEV - 1,)),
            pltpu.SemaphoreType.DMA((N_DEV - 1,)),
            pltpu.SemaphoreType.DMA((N_DEV,)),
            pltpu.SemaphoreType.DMA((N_DEV,)),
            pltpu.SemaphoreType.DMA,
            pltpu.SemaphoreType.DMA,
        ],
        compiler_params=pltpu.CompilerParams(collective_id=0),
    )(x, Wq, K_ext, V_ext, Wo)


# baseline (device time: 57288 ns/iter reference)
import jax
import jax.numpy as jnp
from jax import lax
from jax.experimental import pallas as pl
from jax.experimental.pallas import tpu as pltpu

N_DEV = 4
B_LOC = 2
SQ = 256
HQ = 16
DH = 64
D_MODEL = 512
D_QK = HQ * DH
GCOLS = D_QK // N_DEV
BLK = 64


def kernel(x, Wq, K_ext, V_ext, Wo):
    def body(x_ref, wq_ref, k_hbm, v_hbm, wo_ref, out_ref,
             wq_g, wo_g, k_vmem, v_vmem, x_bf, q_buf, ctx_buf,
             wq_send_sems, wo_send_sems, wq_recv_sems, wo_recv_sems,
             k_sem, v_sem):
        my = lax.axis_index("i")

        b0 = my * B_LOC
        k_copy = pltpu.make_async_copy(k_hbm.at[pl.ds(b0, B_LOC)], k_vmem, k_sem)
        v_copy = pltpu.make_async_copy(v_hbm.at[pl.ds(b0, B_LOC)], v_vmem, v_sem)
        k_copy.start()
        v_copy.start()

        barrier = pltpu.get_barrier_semaphore()
        for d in range(1, N_DEV):
            pl.semaphore_signal(
                barrier, inc=1,
                device_id=((my + d) % N_DEV,),
                device_id_type=pl.DeviceIdType.MESH,
            )
        pl.semaphore_wait(barrier, N_DEV - 1)

        for g in range(N_DEV):
            @pl.when(my == g)
            def _(g=g):
                wq_g[g] = wq_ref[...].astype(jnp.bfloat16)
                wo_g[g] = wo_ref[...].astype(jnp.bfloat16)
                for d in range(1, N_DEV):
                    peer = (g + d) % N_DEV
                    pltpu.make_async_remote_copy(
                        src_ref=wq_g.at[g], dst_ref=wq_g.at[g],
                        send_sem=wq_send_sems.at[d - 1],
                        recv_sem=wq_recv_sems.at[g],
                        device_id=(peer,),
                        device_id_type=pl.DeviceIdType.MESH,
                    ).start()
                    pltpu.make_async_remote_copy(
                        src_ref=wo_g.at[g], dst_ref=wo_g.at[g],
                        send_sem=wo_send_sems.at[d - 1],
                        recv_sem=wo_recv_sems.at[g],
                        device_id=(peer,),
                        device_id_type=pl.DeviceIdType.MESH,
                    ).start()

        x_bf[...] = x_ref[...].reshape(B_LOC * SQ, D_MODEL).astype(jnp.bfloat16)

        k_copy.wait()
        v_copy.wait()

        for g in range(N_DEV):
            @pl.when(my != g)
            def _(g=g):
                pltpu.make_async_remote_copy(
                    src_ref=wq_g.at[g], dst_ref=wq_g.at[g],
                    send_sem=wq_send_sems.at[0],
                    recv_sem=wq_recv_sems.at[g],
                    device_id=(g,), device_id_type=pl.DeviceIdType.MESH,
                ).wait_recv()
                pltpu.make_async_remote_copy(
                    src_ref=wo_g.at[g], dst_ref=wo_g.at[g],
                    send_sem=wo_send_sems.at[0],
                    recv_sem=wo_recv_sems.at[g],
                    device_id=(g,), device_id_type=pl.DeviceIdType.MESH,
                ).wait_recv()

        for g in range(N_DEV):
            q = jnp.dot(x_bf[...], wq_g[g], preferred_element_type=jnp.float32)
            q_buf[:, g * GCOLS:(g + 1) * GCOLS] = q.astype(jnp.bfloat16)

        rows = lax.broadcasted_iota(jnp.int32, (SQ, SQ), 0) // BLK
        cols = lax.broadcasted_iota(jnp.int32, (SQ, SQ), 1) // BLK
        mask = rows == cols
        for b in range(B_LOC):
            for h in range(HQ):
                qh = q_buf[b * SQ:(b + 1) * SQ, h * DH:(h + 1) * DH]
                kh = k_vmem[b, :, h, :].astype(jnp.bfloat16)
                s = lax.dot_general(
                    qh, kh, (((1,), (1,)), ((), ())),
                    preferred_element_type=jnp.float32,
                ) * 0.125
                s = jnp.where(mask, s, -1e9)
                m = jnp.max(s, axis=-1, keepdims=True)
                w = jnp.exp(s - m)
                w = w / jnp.sum(w, axis=-1, keepdims=True)
                vh = v_vmem[b, :, h, :].astype(jnp.bfloat16)
                ctx = jnp.dot(w.astype(jnp.bfloat16), vh,
                              preferred_element_type=jnp.float32)
                ctx_buf[b * SQ:(b + 1) * SQ, h * DH:(h + 1) * DH] = (
                    ctx.astype(jnp.bfloat16))

        acc = jnp.zeros((B_LOC * SQ, D_MODEL), jnp.float32)
        for g in range(N_DEV):
            acc += jnp.dot(ctx_buf[:, g * GCOLS:(g + 1) * GCOLS], wo_g[g],
                           preferred_element_type=jnp.float32)
        for b in range(B_LOC):
            out_ref[b, :, :] = acc[b * SQ:(b + 1) * SQ, :]

        for g in range(N_DEV):
            @pl.when(my == g)
            def _(g=g):
                for d in range(1, N_DEV):
                    peer = (g + d) % N_DEV
                    pltpu.make_async_remote_copy(
                        src_ref=wq_g.at[g], dst_ref=wq_g.at[g],
                        send_sem=wq_send_sems.at[d - 1],
                        recv_sem=wq_recv_sems.at[g],
                        device_id=(peer,),
                        device_id_type=pl.DeviceIdType.MESH,
                    ).wait_send()
                    pltpu.make_async_remote_copy(
                        src_ref=wo_g.at[g], dst_ref=wo_g.at[g],
                        send_sem=wo_send_sems.at[d - 1],
                        recv_sem=wo_recv_sems.at[g],
                        device_id=(peer,),
                        device_id_type=pl.DeviceIdType.MESH,
                    ).wait_send()

    any_space = pl.ANY

    return pl.pallas_call(
        body,
        out_shape=jax.ShapeDtypeStruct((B_LOC, SQ, D_MODEL), jnp.float32),
        in_specs=[
            pl.BlockSpec(memory_space=pltpu.VMEM),
            pl.BlockSpec(memory_space=pltpu.VMEM),
            pl.BlockSpec(memory_space=any_space),
            pl.BlockSpec(memory_space=any_space),
            pl.BlockSpec(memory_space=pltpu.VMEM),
        ],
        out_specs=pl.BlockSpec(memory_space=pltpu.VMEM),
        scratch_shapes=[
            pltpu.VMEM((N_DEV, D_MODEL, GCOLS), jnp.bfloat16),
            pltpu.VMEM((N_DEV, GCOLS, D_MODEL), jnp.bfloat16),
            pltpu.VMEM((B_LOC, SQ, HQ, DH), jnp.float32),
            pltpu.VMEM((B_LOC, SQ, HQ, DH), jnp.float32),
            pltpu.VMEM((B_LOC * SQ, D_MODEL), jnp.bfloat16),
            pltpu.VMEM((B_LOC * SQ, D_QK), jnp.bfloat16),
            pltpu.VMEM((B_LOC * SQ, D_QK), jnp.bfloat16),
            pltpu.SemaphoreType.DMA((N_DEV - 1,)),
            pltpu.SemaphoreType.DMA((N_DEV - 1,)),
            pltpu.SemaphoreType.DMA((N_DEV,)),
            pltpu.SemaphoreType.DMA((N_DEV,)),
            pltpu.SemaphoreType.DMA,
            pltpu.SemaphoreType.DMA,
        ],
        compiler_params=pltpu.CompilerParams(collective_id=0),
    )(x, Wq, K_ext, V_ext, Wo)


# device time: 23466 ns/iter; 2.4413x vs baseline; 2.4413x over previous
import jax
import jax.numpy as jnp
from jax import lax
from jax.experimental import pallas as pl
from jax.experimental.pallas import tpu as pltpu

N_DEV = 4
B_LOC = 2
SQ = 256
HQ = 16
HG = HQ // N_DEV
DH = 64
D_MODEL = 512
D_QK = HQ * DH
GCOLS = D_QK // N_DEV
BLK = 64
CROWS = D_MODEL + D_MODEL


def kernel(x, Wq, K_ext, V_ext, Wo):
    pos = lax.axis_index("i")
    Kl = lax.dynamic_slice_in_dim(K_ext, pos * B_LOC, B_LOC, axis=0)
    Vl = lax.dynamic_slice_in_dim(V_ext, pos * B_LOC, B_LOC, axis=0)
    Kf = Kl.astype(jnp.bfloat16).reshape(B_LOC, SQ, D_QK)
    Vf = Vl.astype(jnp.bfloat16).reshape(B_LOC, SQ, D_QK)
    WoT = jnp.transpose(Wo).astype(jnp.bfloat16)
    x16 = x.astype(jnp.bfloat16)
    Wq16 = Wq.astype(jnp.bfloat16)

    def body(x_ref, wq_ref, kf_ref, vf_ref, wot_ref, out_ref,
             cw, kt_s, vt_s, q_gbuf, ctx_gbuf, acc,
             send_sems, recv_sems, kv_sems):
        my = lax.axis_index("i")

        for g in range(N_DEV):
            @pl.when(my == g)
            def _(g=g):
                cw[g, 0:D_MODEL, :] = wq_ref[...]
                cw[g, D_MODEL:CROWS, :] = wot_ref[...]

        barrier = pltpu.get_barrier_semaphore()
        for d in range(1, N_DEV):
            pl.semaphore_signal(
                barrier, inc=1,
                device_id=((my + d) % N_DEV,),
                device_id_type=pl.DeviceIdType.MESH,
            )
        pl.semaphore_wait(barrier, N_DEV - 1)

        for g in range(N_DEV):
            @pl.when(my == g)
            def _(g=g):
                for d in range(1, N_DEV):
                    peer = (g + d) % N_DEV
                    pltpu.make_async_remote_copy(
                        src_ref=cw.at[g], dst_ref=cw.at[g],
                        send_sem=send_sems.at[d - 1],
                        recv_sem=recv_sems.at[g],
                        device_id=(peer,),
                        device_id_type=pl.DeviceIdType.MESH,
                    ).start()

        kv_copies = []
        for b in range(B_LOC):
            for hp in range(HQ // 2):
                p = b * (HQ // 2) + hp
                c = pltpu.make_async_copy(
                    kf_ref.at[pl.ds(b, 1), :, pl.ds(hp * 2 * DH, 2 * DH)],
                    kt_s.at[pl.ds(p, 1)], kv_sems.at[0])
                c.start()
                kv_copies.append(c)
                c = pltpu.make_async_copy(
                    vf_ref.at[pl.ds(b, 1), :, pl.ds(hp * 2 * DH, 2 * DH)],
                    vt_s.at[pl.ds(p, 1)], kv_sems.at[1])
                c.start()
                kv_copies.append(c)

        rows = lax.broadcasted_iota(jnp.int32, (SQ, SQ), 0) // BLK
        cols = lax.broadcasted_iota(jnp.int32, (SQ, SQ), 1) // BLK
        mask_f = (rows == cols).astype(jnp.float32)

        for c in kv_copies:
            c.wait()

        for j in range(N_DEV):
            g = (my - j) % N_DEV
            if j > 0:
                pltpu.make_async_remote_copy(
                    src_ref=cw.at[g], dst_ref=cw.at[g],
                    send_sem=send_sems.at[0],
                    recv_sem=recv_sems.at[g],
                    device_id=(g,), device_id_type=pl.DeviceIdType.MESH,
                ).wait_recv()

            q_gbuf[...] = (jnp.dot(
                x_ref[...].reshape(B_LOC * SQ, D_MODEL), cw[g, 0:D_MODEL, :],
                preferred_element_type=jnp.float32,
            ) * 0.125).astype(jnp.bfloat16)

            for b in range(B_LOC):
                for hl in range(HG):
                    p = b * (HQ // 2) + g * (HG // 2) + hl // 2
                    lo = (hl % 2) * DH
                    qh = q_gbuf[b * SQ:(b + 1) * SQ, hl * DH:(hl + 1) * DH]
                    s = lax.dot_general(
                        qh, kt_s[p, :, lo:lo + DH], (((1,), (1,)), ((), ())),
                        preferred_element_type=jnp.float32,
                    )
                    e = jnp.exp(s) * mask_f
                    r = 1.0 / jnp.sum(e, axis=-1, keepdims=True)
                    ctx = jnp.dot(e.astype(jnp.bfloat16), vt_s[p, :, lo:lo + DH],
                                  preferred_element_type=jnp.float32) * r
                    ctx_gbuf[b * SQ:(b + 1) * SQ, hl * DH:(hl + 1) * DH] = (
                        ctx.astype(jnp.bfloat16))

            part = lax.dot_general(
                ctx_gbuf[...], cw[g, D_MODEL:CROWS, :],
                (((1,), (1,)), ((), ())),
                preferred_element_type=jnp.float32,
            )
            if j == 0:
                acc[...] = part
            else:
                acc[...] = acc[...] + part

        out_ref[...] = acc[...].astype(jnp.bfloat16)

        for g in range(N_DEV):
            @pl.when(my == g)
            def _(g=g):
                for d in range(1, N_DEV):
                    peer = (g + d) % N_DEV
                    pltpu.make_async_remote_copy(
                        src_ref=cw.at[g], dst_ref=cw.at[g],
                        send_sem=send_sems.at[d - 1],
                        recv_sem=recv_sems.at[g],
                        device_id=(peer,),
                        device_id_type=pl.DeviceIdType.MESH,
                    ).wait_send()

    out2d = pl.pallas_call(
        body,
        out_shape=jax.ShapeDtypeStruct((B_LOC * SQ, D_MODEL), jnp.bfloat16),
        in_specs=[
            pl.BlockSpec(memory_space=pltpu.VMEM),
            pl.BlockSpec(memory_space=pltpu.VMEM),
            pl.BlockSpec(memory_space=pltpu.VMEM),
            pl.BlockSpec(memory_space=pltpu.VMEM),
            pl.BlockSpec(memory_space=pltpu.VMEM),
        ],
        out_specs=pl.BlockSpec(memory_space=pltpu.VMEM),
        scratch_shapes=[
            pltpu.VMEM((N_DEV, CROWS, GCOLS), jnp.bfloat16),
            pltpu.VMEM((B_LOC * HQ // 2, SQ, 2 * DH), jnp.bfloat16),
            pltpu.VMEM((B_LOC * HQ // 2, SQ, 2 * DH), jnp.bfloat16),
            pltpu.VMEM((B_LOC * SQ, GCOLS), jnp.bfloat16),
            pltpu.VMEM((B_LOC * SQ, GCOLS), jnp.bfloat16),
            pltpu.VMEM((B_LOC * SQ, D_MODEL), jnp.float32),
            pltpu.SemaphoreType.DMA((N_DEV - 1,)),
            pltpu.SemaphoreType.DMA((N_DEV,)),
            pltpu.SemaphoreType.DMA((2,)),
        ],
        compiler_params=pltpu.CompilerParams(collective_id=0),
    )(x16, Wq16, Kf, Vf, WoT)
    return out2d.reshape(B_LOC, SQ, D_MODEL)
